# baseline (speedup 1.0000x reference)
import functools

import jax
import jax.numpy as jnp
from jax import lax
from jax.experimental import pallas as pl
from jax.experimental.pallas import tpu as pltpu
from jax.experimental.pallas import tpu_sc as plsc

N_USERS = 4000
N_ITEMS = 6000
N_NODES = N_USERS + N_ITEMS
N_EDGES = 320000
DIM = 128
HALF = 64
N_LAYERS = 3
BATCH_USERS = 1024

NC = 2
NS = 16
K = 128

N_PAD = 10240
ROWS_PT = N_PAD // NS
SB = 8
NSB = 20
CHUNKS_PT = SB * NSB
E_PAD = NS * CHUNKS_PT * K
NIX = 2
ITEMS_PAD = 6016
IROWS_PT = ITEMS_PAD // NS
U_PT = BATCH_USERS // NS


def _propagate_body(src2, dst2, w2, emb0, upair, light, uout, e1buf, e2buf,
                    accum, srcv, didxv, wv, rows, stage,
                    sem_idx, sem_g, sem_s, sem):
    cid = lax.axis_index("c")
    sid = lax.axis_index("s")
    rbase = sid * ROWS_PT
    ebase = sid * CHUNKS_PT

    def zero_stage():
        @pl.loop(0, K)
        def _z(r):
            for c in range(HALF // 16):
                stage[r, pl.ds(c * 16, 16)] = jnp.zeros((16,), jnp.float32)

    zero_stage()

    sources = [emb0, e1buf, e2buf]
    for layer in range(N_LAYERS):
        gather_src = sources[layer]
        for b in range(ROWS_PT // K):
            pltpu.sync_copy(stage, accum.at[pl.ds(rbase + b * K, K)])
        plsc.subcore_barrier()

        gsrc = gather_src

        def fire_idx(s):
            ib = lax.rem(s, NIX)
            row = ebase + s * SB
            pltpu.async_copy(src2.at[cid, pl.ds(row, SB)], sidx.at[ib],
                             sem_idx)
            pltpu.async_copy(dst2.at[pl.ds(row, SB)], didx.at[ib], sem_idx)
            pltpu.async_copy(w2.at[pl.ds(row, SB)], wblk.at[ib], sem_idx)

        def drain_idx():
            pltpu.make_async_copy(src2.at[cid, pl.ds(ebase, SB)],
                                  sidx.at[0], sem_idx).wait()
            pltpu.make_async_copy(dst2.at[pl.ds(ebase, SB)], didx.at[0],
                                  sem_idx).wait()
            pltpu.make_async_copy(w2.at[pl.ds(ebase, SB)], wblk.at[0],
                                  sem_idx).wait()

        @pl.loop(0, CHUNKS_PT)
        def _chunk(g):
            row = ebase + g
            pltpu.sync_copy(src2.at[cid, row], srcv)
            pltpu.sync_copy(dst2.at[row], didxv)
            pltpu.sync_copy(w2.at[row], wv)
            pltpu.async_copy(gsrc.at[srcv], rows.at[0], sem_g).wait()

            @pl.loop(0, K // 16)
            def _scale(gg):
                w16 = wv[pl.ds(gg * 16, 16)]
                for t in range(16):
                    r = gg * 16 + t
                    wt = w16[t]
                    for c in range(HALF // 16):
                        sl = pl.ds(c * 16, 16)
                        rows[0, r, sl] = rows[0, r, sl] * wt

            pltpu.sync_copy(rows.at[0], accum.at[didxv], add=True)

        plsc.subcore_barrier()

        if layer < N_LAYERS - 1:
            nxt = sources[layer + 1]
            for b in range(ROWS_PT // K):
                sl = pl.ds(rbase + b * K, K)
                hsl = pl.ds(cid * N_PAD + rbase + b * K, K)
                pltpu.sync_copy(accum.at[sl], rows.at[0])
                pltpu.sync_copy(rows.at[0], nxt.at[hsl])
            plsc.subcore_barrier()

    for b in range(ROWS_PT // K):
        sl = pl.ds(rbase + b * K, K)
        hsl = pl.ds(cid * N_PAD + rbase + b * K, K)
        pltpu.sync_copy(emb0.at[hsl], rows.at[0])
        pltpu.sync_copy(e1buf.at[hsl], rows.at[1])
        pltpu.sync_copy(e2buf.at[hsl], rows.at[2])
        pltpu.sync_copy(accum.at[sl], rows.at[3])

        @pl.loop(0, K)
        def _combine(r):
            for c in range(HALF // 16):
                cs = pl.ds(c * 16, 16)
                acc = (rows[0, r, cs] + rows[1, r, cs]
                       + rows[2, r, cs] + rows[3, r, cs])
                rows[0, r, cs] = acc * 0.25

        pltpu.sync_copy(rows.at[0], light.at[hsl])
    plsc.subcore_barrier()

    pltpu.sync_copy(upair.at[cid, pl.ds(sid * U_PT, U_PT)],
                    srcv.at[pl.ds(0, U_PT)])
    pltpu.async_copy(light.at[srcv.at[pl.ds(0, U_PT)]],
                     rows.at[0, pl.ds(0, U_PT)], sem).wait()
    pltpu.sync_copy(rows.at[0, pl.ds(0, U_PT)],
                    uout.at[pl.ds(cid * BATCH_USERS + sid * U_PT, U_PT)])


@jax.jit
def _propagate(src2, dst2, w2, emb0, upair):
    mesh = plsc.VectorSubcoreMesh(core_axis_name="c", subcore_axis_name="s")
    return pl.kernel(
        _propagate_body,
        out_type=(
            jax.ShapeDtypeStruct((NC * N_PAD, HALF), jnp.float32),
            jax.ShapeDtypeStruct((NC * BATCH_USERS, HALF), jnp.float32),
            jax.ShapeDtypeStruct((NC * N_PAD, HALF), jnp.float32),
            jax.ShapeDtypeStruct((NC * N_PAD, HALF), jnp.float32),
        ),
        mesh=mesh,
        scratch_types=[
            pltpu.VMEM_SHARED((N_PAD, HALF), jnp.float32),
            pltpu.VMEM((K,), jnp.int32),
            pltpu.VMEM((K,), jnp.int32),
            pltpu.VMEM((K,), jnp.float32),
            pltpu.VMEM((SB, K, HALF), jnp.float32),
            pltpu.VMEM((K, HALF), jnp.float32),
            pltpu.SemaphoreType.DMA,
            pltpu.SemaphoreType.DMA,
            pltpu.SemaphoreType.DMA,
            pltpu.SemaphoreType.DMA,
        ],
        compiler_params=pltpu.CompilerParams(use_tc_tiling_on_sc=False),
    )(src2, dst2, w2, emb0, upair)


def _rating_body(u0, u1, i0, i1, out):
    acc = jax.lax.dot_general(u0[...], i0[...], (((1,), (1,)), ((), ())),
                              preferred_element_type=jnp.float32)
    acc += jax.lax.dot_general(u1[...], i1[...], (((1,), (1,)), ((), ())),
                               preferred_element_type=jnp.float32)
    out[...] = 1.0 / (1.0 + jnp.exp(-acc))


@jax.jit
def _rating(u0, u1, i0, i1):
    m_blk = 128
    grid = (BATCH_USERS // m_blk,)
    return pl.pallas_call(
        _rating_body,
        grid=grid,
        in_specs=[
            pl.BlockSpec((m_blk, HALF), lambda i: (i, 0)),
            pl.BlockSpec((m_blk, HALF), lambda i: (i, 0)),
            pl.BlockSpec((ITEMS_PAD, HALF), lambda i: (0, 0)),
            pl.BlockSpec((ITEMS_PAD, HALF), lambda i: (0, 0)),
        ],
        out_specs=pl.BlockSpec((m_blk, ITEMS_PAD), lambda i: (i, 0)),
        out_shape=jax.ShapeDtypeStruct((BATCH_USERS, ITEMS_PAD), jnp.float32),
    )(u0, u1, i0, i1)


def kernel(user_emb, item_emb, edge_weight, edge_index, users):
    all_emb = jnp.concatenate([user_emb, item_emb], axis=0)
    all_emb = jnp.pad(all_emb, ((0, N_PAD - N_NODES), (0, 0)))
    emb0 = all_emb.reshape(N_PAD, NC, HALF).transpose(1, 0, 2)
    emb0 = emb0.reshape(NC * N_PAD, HALF)

    src = edge_index[0]
    dst = edge_index[1]
    pad_e = E_PAD - N_EDGES
    pad_rows = N_NODES + (jnp.arange(pad_e, dtype=jnp.int32)
                          % (N_PAD - N_NODES))
    src_p = jnp.concatenate([src, pad_rows])
    dst_p = jnp.concatenate([dst, pad_rows])
    w_p = jnp.concatenate([edge_weight, jnp.zeros((pad_e,), jnp.float32)])
    src2 = jnp.stack([src_p, src_p + N_PAD]).reshape(NC, -1, K)
    dst2 = dst_p.reshape(-1, K)
    w2 = w_p.reshape(-1, K)
    upair = jnp.stack([users, users + N_PAD])

    light, uout, _, _ = _propagate(src2, dst2, w2, emb0, upair)

    i0 = light[N_USERS:N_USERS + ITEMS_PAD]
    i1 = light[N_PAD + N_USERS:N_PAD + N_USERS + ITEMS_PAD]
    u0 = uout[:BATCH_USERS]
    u1 = uout[BATCH_USERS:]
    rating = _rating(u0, u1, i0, i1)
    return rating[:, :N_ITEMS]


# speedup vs baseline: 3.0798x; 3.0798x over previous
import functools

import jax
import jax.numpy as jnp
from jax import lax
from jax.experimental import pallas as pl
from jax.experimental.pallas import tpu as pltpu
from jax.experimental.pallas import tpu_sc as plsc

N_USERS = 4000
N_ITEMS = 6000
N_NODES = N_USERS + N_ITEMS
N_EDGES = 320000
DIM = 128
HALF = 64
N_LAYERS = 3
BATCH_USERS = 1024

NC = 2
NS = 16
K = 128

N_PAD = 10240
ROWS_PT = N_PAD // NS
SB = 8
NSB = 20
CHUNKS_PT = SB * NSB
E_PAD = NS * CHUNKS_PT * K
NIX = 2
ITEMS_PAD = 6016
IROWS_PT = ITEMS_PAD // NS
U_PT = BATCH_USERS // NS


def _propagate_body(src2, dst2, w2, emb0, upair, light, uout, e1buf, e2buf,
                    accum, srcv, sidx, didx, wblk,
                    dv0, dv1, dv2, dv3, dv4, dv5, dv6, dv7,
                    rows, stage, sem_ia, sem_da, sem_wa, sem_g, sem_s,
                    sem):
    cid = lax.axis_index("c")
    sid = lax.axis_index("s")
    rbase = sid * ROWS_PT
    ebase = sid * CHUNKS_PT

    def zero_stage():
        @pl.loop(0, K)
        def _z(r):
            for c in range(HALF // 16):
                stage[r, pl.ds(c * 16, 16)] = jnp.zeros((16,), jnp.float32)

    zero_stage()

    sources = [emb0, e1buf, e2buf]
    for layer in range(N_LAYERS):
        gather_src = sources[layer]
        for b in range(ROWS_PT // K):
            pltpu.sync_copy(stage, accum.at[pl.ds(rbase + b * K, K)])
        plsc.subcore_barrier()

        gsrc = gather_src
        dvs = [dv0, dv1, dv2, dv3, dv4, dv5, dv6, dv7]

        @pl.loop(0, NSB)
        def _superblock(t):
            row = ebase + t * SB
            i1 = pltpu.async_copy(src2.at[cid, pl.ds(row, SB)], sidx, sem_ia)
            i2 = pltpu.async_copy(dst2.at[pl.ds(row, SB)], didx, sem_da)
            i3 = pltpu.async_copy(w2.at[pl.ds(row, SB)], wblk, sem_wa)
            i1.wait()
            gath = [
                pltpu.async_copy(gsrc.at[sidx.at[j]], rows.at[j],
                                 sem_g.at[j])
                for j in range(SB)
            ]
            i2.wait()
            i3.wait()
            scat = []
            for j in range(SB):
                for c in range(K // 16):
                    cs = pl.ds(c * 16, 16)
                    dvs[j][cs] = didx[j, cs]
                gath[j].wait()

                @pl.loop(0, K // 16)
                def _scale(gg):
                    w16 = wblk[j, pl.ds(gg * 16, 16)]
                    for tt in range(16):
                        r = gg * 16 + tt
                        wt = w16[tt]
                        for c in range(HALF // 16):
                            sl = pl.ds(c * 16, 16)
                            rows[j, r, sl] = rows[j, r, sl] * wt

                scat.append(
                    pltpu.async_copy(rows.at[j], accum.at[dvs[j]],
                                     sem_s.at[j], add=True))
            for d in scat:
                d.wait()

        plsc.subcore_barrier()

        if layer < N_LAYERS - 1:
            nxt = sources[layer + 1]
            for b in range(ROWS_PT // K):
                sl = pl.ds(rbase + b * K, K)
                hsl = pl.ds(cid * N_PAD + rbase + b * K, K)
                pltpu.sync_copy(accum.at[sl], rows.at[0])
                pltpu.sync_copy(rows.at[0], nxt.at[hsl])
            plsc.subcore_barrier()

    for b in range(ROWS_PT // K):
        sl = pl.ds(rbase + b * K, K)
        hsl = pl.ds(cid * N_PAD + rbase + b * K, K)
        pltpu.sync_copy(emb0.at[hsl], rows.at[0])
        pltpu.sync_copy(e1buf.at[hsl], rows.at[1])
        pltpu.sync_copy(e2buf.at[hsl], rows.at[2])
        pltpu.sync_copy(accum.at[sl], rows.at[3])

        @pl.loop(0, K)
        def _combine(r):
            for c in range(HALF // 16):
                cs = pl.ds(c * 16, 16)
                acc = (rows[0, r, cs] + rows[1, r, cs]
                       + rows[2, r, cs] + rows[3, r, cs])
                rows[0, r, cs] = acc * 0.25

        pltpu.sync_copy(rows.at[0], light.at[hsl])
    plsc.subcore_barrier()

    pltpu.sync_copy(upair.at[cid, pl.ds(sid * U_PT, U_PT)],
                    srcv.at[pl.ds(0, U_PT)])
    pltpu.async_copy(light.at[srcv.at[pl.ds(0, U_PT)]],
                     rows.at[0, pl.ds(0, U_PT)], sem).wait()
    pltpu.sync_copy(rows.at[0, pl.ds(0, U_PT)],
                    uout.at[pl.ds(cid * BATCH_USERS + sid * U_PT, U_PT)])


@jax.jit
def _propagate(src2, dst2, w2, emb0, upair):
    mesh = plsc.VectorSubcoreMesh(core_axis_name="c", subcore_axis_name="s")
    return pl.kernel(
        _propagate_body,
        out_type=(
            jax.ShapeDtypeStruct((NC * N_PAD, HALF), jnp.float32),
            jax.ShapeDtypeStruct((NC * BATCH_USERS, HALF), jnp.float32),
            jax.ShapeDtypeStruct((NC * N_PAD, HALF), jnp.float32),
            jax.ShapeDtypeStruct((NC * N_PAD, HALF), jnp.float32),
        ),
        mesh=mesh,
        scratch_types=[
            pltpu.VMEM_SHARED((N_PAD, HALF), jnp.float32),
            pltpu.VMEM((K,), jnp.int32),
            pltpu.VMEM((SB, K), jnp.int32),
            pltpu.VMEM((SB, K), jnp.int32),
            pltpu.VMEM((SB, K), jnp.float32),
            pltpu.VMEM((K,), jnp.int32),
            pltpu.VMEM((K,), jnp.int32),
            pltpu.VMEM((K,), jnp.int32),
            pltpu.VMEM((K,), jnp.int32),
            pltpu.VMEM((K,), jnp.int32),
            pltpu.VMEM((K,), jnp.int32),
            pltpu.VMEM((K,), jnp.int32),
            pltpu.VMEM((K,), jnp.int32),
            pltpu.VMEM((SB, K, HALF), jnp.float32),
            pltpu.VMEM((K, HALF), jnp.float32),
            pltpu.SemaphoreType.DMA,
            pltpu.SemaphoreType.DMA,
            pltpu.SemaphoreType.DMA,
            pltpu.SemaphoreType.DMA((SB,)),
            pltpu.SemaphoreType.DMA((SB,)),
            pltpu.SemaphoreType.DMA,
        ],
        compiler_params=pltpu.CompilerParams(use_tc_tiling_on_sc=False),
    )(src2, dst2, w2, emb0, upair)


def _rating_body(u0, u1, i0, i1, out):
    acc = jax.lax.dot_general(u0[...], i0[...], (((1,), (1,)), ((), ())),
                              preferred_element_type=jnp.float32)
    acc += jax.lax.dot_general(u1[...], i1[...], (((1,), (1,)), ((), ())),
                               preferred_element_type=jnp.float32)
    out[...] = 1.0 / (1.0 + jnp.exp(-acc))


@jax.jit
def _rating(u0, u1, i0, i1):
    m_blk = 128
    grid = (BATCH_USERS // m_blk,)
    return pl.pallas_call(
        _rating_body,
        grid=grid,
        in_specs=[
            pl.BlockSpec((m_blk, HALF), lambda i: (i, 0)),
            pl.BlockSpec((m_blk, HALF), lambda i: (i, 0)),
            pl.BlockSpec((ITEMS_PAD, HALF), lambda i: (0, 0)),
            pl.BlockSpec((ITEMS_PAD, HALF), lambda i: (0, 0)),
        ],
        out_specs=pl.BlockSpec((m_blk, ITEMS_PAD), lambda i: (i, 0)),
        out_shape=jax.ShapeDtypeStruct((BATCH_USERS, ITEMS_PAD), jnp.float32),
    )(u0, u1, i0, i1)


def kernel(user_emb, item_emb, edge_weight, edge_index, users):
    all_emb = jnp.concatenate([user_emb, item_emb], axis=0)
    all_emb = jnp.pad(all_emb, ((0, N_PAD - N_NODES), (0, 0)))
    emb0 = all_emb.reshape(N_PAD, NC, HALF).transpose(1, 0, 2)
    emb0 = emb0.reshape(NC * N_PAD, HALF)

    src = edge_index[0]
    dst = edge_index[1]
    pad_e = E_PAD - N_EDGES
    pad_rows = N_NODES + (jnp.arange(pad_e, dtype=jnp.int32)
                          % (N_PAD - N_NODES))
    src_p = jnp.concatenate([src, pad_rows])
    dst_p = jnp.concatenate([dst, pad_rows])
    w_p = jnp.concatenate([edge_weight, jnp.zeros((pad_e,), jnp.float32)])
    src2 = jnp.stack([src_p, src_p + N_PAD]).reshape(NC, -1, K)
    dst2 = dst_p.reshape(-1, K)
    w2 = w_p.reshape(-1, K)
    upair = jnp.stack([users, users + N_PAD])

    light, uout, _, _ = _propagate(src2, dst2, w2, emb0, upair)

    i0 = light[N_USERS:N_USERS + ITEMS_PAD]
    i1 = light[N_PAD + N_USERS:N_PAD + N_USERS + ITEMS_PAD]
    u0 = uout[:BATCH_USERS]
    u1 = uout[BATCH_USERS:]
    rating = _rating(u0, u1, i0, i1)
    return rating[:, :N_ITEMS]


# speedup vs baseline: 4.1384x; 1.3437x over previous
import functools

import jax
import jax.numpy as jnp
from jax import lax
from jax.experimental import pallas as pl
from jax.experimental.pallas import tpu as pltpu
from jax.experimental.pallas import tpu_sc as plsc

N_USERS = 4000
N_ITEMS = 6000
N_NODES = N_USERS + N_ITEMS
N_EDGES = 320000
DIM = 128
HALF = 64
N_LAYERS = 3
BATCH_USERS = 1024

NC = 2
NS = 16
K = 128

N_PAD = 10240
ROWS_PT = N_PAD // NS
SB = 8
NSB = 20
CHUNKS_PT = SB * NSB
E_PAD = NS * CHUNKS_PT * K
NIX = 2
ITEMS_PAD = 6016
IROWS_PT = ITEMS_PAD // NS
U_PT = BATCH_USERS // NS


def _propagate_body(src2, dst2, w2, emb0, upair, light, uout, e1buf, e2buf,
                    accum, srcv, sidx, didx, wblk,
                    dv0, dv1, dv2, dv3, dv4, dv5, dv6, dv7,
                    rows, stage, sem_ia, sem_da, sem_wa, sem_g, sem_s,
                    sem):
    cid = lax.axis_index("c")
    sid = lax.axis_index("s")
    rbase = sid * ROWS_PT
    ebase = sid * CHUNKS_PT

    def zero_stage():
        @pl.loop(0, K)
        def _z(r):
            for c in range(HALF // 16):
                stage[r, pl.ds(c * 16, 16)] = jnp.zeros((16,), jnp.float32)

    zero_stage()

    sources = [emb0, e1buf, e2buf]
    for layer in range(N_LAYERS):
        gather_src = sources[layer]
        za = [
            pltpu.async_copy(stage, accum.at[pl.ds(rbase + b * K, K)],
                             sem_g.at[b])
            for b in range(ROWS_PT // K)
        ]
        for d in za:
            d.wait()
        plsc.subcore_barrier()

        gsrc = gather_src
        dvs = [dv0, dv1, dv2, dv3, dv4, dv5, dv6, dv7]

        @pl.loop(0, NSB)
        def _superblock(t):
            row = ebase + t * SB
            i1 = pltpu.async_copy(src2.at[cid, pl.ds(row, SB)], sidx, sem_ia)
            i2 = pltpu.async_copy(dst2.at[pl.ds(row, SB)], didx, sem_da)
            i3 = pltpu.async_copy(w2.at[pl.ds(row, SB)], wblk, sem_wa)
            i1.wait()
            gath = [
                pltpu.async_copy(gsrc.at[sidx.at[j]], rows.at[j],
                                 sem_g.at[j])
                for j in range(SB)
            ]
            i2.wait()
            i3.wait()
            scat = []
            for j in range(SB):
                for c in range(K // 16):
                    cs = pl.ds(c * 16, 16)
                    dvs[j][cs] = didx[j, cs]
                gath[j].wait()

                @pl.loop(0, K // 16, unroll=2)
                def _scale(gg):
                    w16 = wblk[j, pl.ds(gg * 16, 16)]
                    for tt in range(16):
                        r = gg * 16 + tt
                        wt = w16[tt]
                        for c in range(HALF // 16):
                            sl = pl.ds(c * 16, 16)
                            rows[j, r, sl] = rows[j, r, sl] * wt

                scat.append(
                    pltpu.async_copy(rows.at[j], accum.at[dvs[j]],
                                     sem_s.at[j], add=True))
            for d in scat:
                d.wait()

        plsc.subcore_barrier()

        if layer < N_LAYERS - 1:
            nxt = sources[layer + 1]
            nb = ROWS_PT // K
            lds = [
                pltpu.async_copy(
                    accum.at[pl.ds(rbase + b * K, K)], rows.at[b],
                    sem_g.at[b])
                for b in range(nb)
            ]
            sts = []
            for b in range(nb):
                lds[b].wait()
                sts.append(pltpu.async_copy(
                    rows.at[b],
                    nxt.at[pl.ds(cid * N_PAD + rbase + b * K, K)],
                    sem_s.at[b]))
            for d in sts:
                d.wait()
            plsc.subcore_barrier()

    nb = ROWS_PT // K
    srcs4 = [emb0, e1buf, e2buf]
    loads = {}

    def fire_block(b):
        base = 4 * (b % 2)
        hsl = pl.ds(cid * N_PAD + rbase + b * K, K)
        ds4 = [pltpu.async_copy(srcs4[i].at[hsl], rows.at[base + i],
                                sem_g.at[base + i]) for i in range(3)]
        ds4.append(pltpu.async_copy(accum.at[pl.ds(rbase + b * K, K)],
                                    rows.at[base + 3], sem_g.at[base + 3]))
        loads[b] = ds4

    fire_block(0)
    sts = {}
    for b in range(nb):
        if b + 1 < nb and b + 1 not in loads:
            pass
        for d in loads[b]:
            d.wait()
        base = 4 * (b % 2)

        @pl.loop(0, K)
        def _combine(r):
            for c in range(HALF // 16):
                cs = pl.ds(c * 16, 16)
                acc = (rows[base + 0, r, cs] + rows[base + 1, r, cs]
                       + rows[base + 2, r, cs] + rows[base + 3, r, cs])
                rows[base + 0, r, cs] = acc * 0.25

        if b + 2 in sts:
            pass
        sts[b] = pltpu.async_copy(
            rows.at[base],
            light.at[pl.ds(cid * N_PAD + rbase + b * K, K)], sem_s.at[b % 2])
        if b + 1 < nb:
            if b - 1 in sts:
                sts[b - 1].wait()
            fire_block(b + 1)
    sts[nb - 1].wait()
    plsc.subcore_barrier()

    pltpu.sync_copy(upair.at[cid, pl.ds(sid * U_PT, U_PT)],
                    srcv.at[pl.ds(0, U_PT)])
    pltpu.async_copy(light.at[srcv.at[pl.ds(0, U_PT)]],
                     rows.at[0, pl.ds(0, U_PT)], sem).wait()
    pltpu.sync_copy(rows.at[0, pl.ds(0, U_PT)],
                    uout.at[pl.ds(cid * BATCH_USERS + sid * U_PT, U_PT)])


@jax.jit
def _propagate(src2, dst2, w2, emb0, upair):
    mesh = plsc.VectorSubcoreMesh(core_axis_name="c", subcore_axis_name="s")
    return pl.kernel(
        _propagate_body,
        out_type=(
            jax.ShapeDtypeStruct((NC * N_PAD, HALF), jnp.float32),
            jax.ShapeDtypeStruct((NC * BATCH_USERS, HALF), jnp.float32),
            jax.ShapeDtypeStruct((NC * N_PAD, HALF), jnp.float32),
            jax.ShapeDtypeStruct((NC * N_PAD, HALF), jnp.float32),
        ),
        mesh=mesh,
        scratch_types=[
            pltpu.VMEM_SHARED((N_PAD, HALF), jnp.float32),
            pltpu.VMEM((K,), jnp.int32),
            pltpu.VMEM((SB, K), jnp.int32),
            pltpu.VMEM((SB, K), jnp.int32),
            pltpu.VMEM((SB, K), jnp.float32),
            pltpu.VMEM((K,), jnp.int32),
            pltpu.VMEM((K,), jnp.int32),
            pltpu.VMEM((K,), jnp.int32),
            pltpu.VMEM((K,), jnp.int32),
            pltpu.VMEM((K,), jnp.int32),
            pltpu.VMEM((K,), jnp.int32),
            pltpu.VMEM((K,), jnp.int32),
            pltpu.VMEM((K,), jnp.int32),
            pltpu.VMEM((SB, K, HALF), jnp.float32),
            pltpu.VMEM((K, HALF), jnp.float32),
            pltpu.SemaphoreType.DMA,
            pltpu.SemaphoreType.DMA,
            pltpu.SemaphoreType.DMA,
            pltpu.SemaphoreType.DMA((SB,)),
            pltpu.SemaphoreType.DMA((SB,)),
            pltpu.SemaphoreType.DMA,
        ],
        compiler_params=pltpu.CompilerParams(use_tc_tiling_on_sc=False),
    )(src2, dst2, w2, emb0, upair)


def _rating_body(u0, u1, i0, i1, out):
    acc = jax.lax.dot_general(u0[...], i0[...], (((1,), (1,)), ((), ())),
                              preferred_element_type=jnp.float32)
    acc += jax.lax.dot_general(u1[...], i1[...], (((1,), (1,)), ((), ())),
                               preferred_element_type=jnp.float32)
    out[...] = 1.0 / (1.0 + jnp.exp(-acc))


@jax.jit
def _rating(u0, u1, i0, i1):
    m_blk = 128
    grid = (BATCH_USERS // m_blk,)
    return pl.pallas_call(
        _rating_body,
        grid=grid,
        in_specs=[
            pl.BlockSpec((m_blk, HALF), lambda i: (i, 0)),
            pl.BlockSpec((m_blk, HALF), lambda i: (i, 0)),
            pl.BlockSpec((ITEMS_PAD, HALF), lambda i: (0, 0)),
            pl.BlockSpec((ITEMS_PAD, HALF), lambda i: (0, 0)),
        ],
        out_specs=pl.BlockSpec((m_blk, ITEMS_PAD), lambda i: (i, 0)),
        out_shape=jax.ShapeDtypeStruct((BATCH_USERS, ITEMS_PAD), jnp.float32),
    )(u0, u1, i0, i1)


def kernel(user_emb, item_emb, edge_weight, edge_index, users):
    all_emb = jnp.concatenate([user_emb, item_emb], axis=0)
    all_emb = jnp.pad(all_emb, ((0, N_PAD - N_NODES), (0, 0)))
    emb0 = all_emb.reshape(N_PAD, NC, HALF).transpose(1, 0, 2)
    emb0 = emb0.reshape(NC * N_PAD, HALF)

    src = edge_index[0]
    dst = edge_index[1]
    pad_e = E_PAD - N_EDGES
    pad_rows = N_NODES + (jnp.arange(pad_e, dtype=jnp.int32)
                          % (N_PAD - N_NODES))
    src_p = jnp.concatenate([src, pad_rows])
    dst_p = jnp.concatenate([dst, pad_rows])
    w_p = jnp.concatenate([edge_weight, jnp.zeros((pad_e,), jnp.float32)])
    src2 = jnp.stack([src_p, src_p + N_PAD]).reshape(NC, -1, K)
    dst2 = dst_p.reshape(-1, K)
    w2 = w_p.reshape(-1, K)
    upair = jnp.stack([users, users + N_PAD])

    light, uout, _, _ = _propagate(src2, dst2, w2, emb0, upair)

    i0 = light[N_USERS:N_USERS + ITEMS_PAD]
    i1 = light[N_PAD + N_USERS:N_PAD + N_USERS + ITEMS_PAD]
    u0 = uout[:BATCH_USERS]
    u1 = uout[BATCH_USERS:]
    rating = _rating(u0, u1, i0, i1)
    return rating[:, :N_ITEMS]


# speedup vs baseline: 4.1948x; 1.0136x over previous
import functools

import jax
import jax.numpy as jnp
from jax import lax
from jax.experimental import pallas as pl
from jax.experimental.pallas import tpu as pltpu
from jax.experimental.pallas import tpu_sc as plsc

N_USERS = 4000
N_ITEMS = 6000
N_NODES = N_USERS + N_ITEMS
N_EDGES = 320000
DIM = 128
HALF = 64
N_LAYERS = 3
BATCH_USERS = 1024

NC = 2
NS = 16
K = 128

N_PAD = 10240
ROWS_PT = N_PAD // NS
SB = 8
NSB = 20
CHUNKS_PT = SB * NSB
E_PAD = NS * CHUNKS_PT * K
NIX = 2
ITEMS_PAD = 6016
IROWS_PT = ITEMS_PAD // NS
U_PT = BATCH_USERS // NS


def _propagate_body(src2, dst2, w2, emb0, upair, light, uout, e1buf, e2buf,
                    accum, srcv, sidx, didx, wblk,
                    dv0, dv1, dv2, dv3, dv4, dv5, dv6, dv7,
                    sv0, sv1, sv2, sv3, sv4, sv5, sv6, sv7,
                    rows, stage, sem_ia, sem_da, sem_wa, sem_g, sem_s,
                    sem):
    cid = lax.axis_index("c")
    sid = lax.axis_index("s")
    rbase = sid * ROWS_PT
    ebase = sid * CHUNKS_PT

    def zero_stage():
        @pl.loop(0, K)
        def _z(r):
            for c in range(HALF // 16):
                stage[r, pl.ds(c * 16, 16)] = jnp.zeros((16,), jnp.float32)

    zero_stage()

    sources = [emb0, e1buf, e2buf]
    for layer in range(N_LAYERS):
        gather_src = sources[layer]
        za = [
            pltpu.async_copy(stage, accum.at[pl.ds(rbase + b * K, K)],
                             sem_g.at[b])
            for b in range(ROWS_PT // K)
        ]
        for d in za:
            d.wait()
        plsc.subcore_barrier()

        gsrc = gather_src
        dvs = [dv0, dv1, dv2, dv3, dv4, dv5, dv6, dv7]
        svs = [sv0, sv1, sv2, sv3, sv4, sv5, sv6, sv7]

        @pl.loop(0, NSB)
        def _superblock(t):
            row = ebase + t * SB
            i1 = pltpu.async_copy(src2.at[pl.ds(row, SB)], sidx, sem_ia)
            i2 = pltpu.async_copy(dst2.at[pl.ds(row, SB)], didx, sem_da)
            i3 = pltpu.async_copy(w2.at[pl.ds(row, SB)], wblk, sem_wa)
            i1.wait()
            off = cid * N_PAD
            gath = []
            for j in range(SB):
                for c in range(K // 16):
                    cs = pl.ds(c * 16, 16)
                    svs[j][cs] = sidx[j, cs] + off
                gath.append(
                    pltpu.async_copy(gsrc.at[svs[j]], rows.at[j],
                                     sem_g.at[j]))
            i2.wait()
            i3.wait()
            scat = []
            for j in range(SB):
                for c in range(K // 16):
                    cs = pl.ds(c * 16, 16)
                    dvs[j][cs] = didx[j, cs]
                gath[j].wait()

                @pl.loop(0, K // 16, unroll=2)
                def _scale(gg):
                    w16 = wblk[j, pl.ds(gg * 16, 16)]
                    for tt in range(16):
                        r = gg * 16 + tt
                        wt = w16[tt]
                        for c in range(HALF // 16):
                            sl = pl.ds(c * 16, 16)
                            rows[j, r, sl] = rows[j, r, sl] * wt

                scat.append(
                    pltpu.async_copy(rows.at[j], accum.at[dvs[j]],
                                     sem_s.at[j], add=True))
            for d in scat:
                d.wait()

        plsc.subcore_barrier()

        if layer < N_LAYERS - 1:
            nxt = sources[layer + 1]
            nb = ROWS_PT // K
            lds = [
                pltpu.async_copy(
                    accum.at[pl.ds(rbase + b * K, K)], rows.at[b],
                    sem_g.at[b])
                for b in range(nb)
            ]
            sts = []
            for b in range(nb):
                lds[b].wait()
                sts.append(pltpu.async_copy(
                    rows.at[b],
                    nxt.at[pl.ds(cid * N_PAD + rbase + b * K, K)],
                    sem_s.at[b]))
            for d in sts:
                d.wait()
            plsc.subcore_barrier()

    nb = ROWS_PT // K
    srcs4 = [emb0, e1buf, e2buf]
    loads = {}

    def fire_block(b):
        base = 4 * (b % 2)
        hsl = pl.ds(cid * N_PAD + rbase + b * K, K)
        ds4 = [pltpu.async_copy(srcs4[i].at[hsl], rows.at[base + i],
                                sem_g.at[base + i]) for i in range(3)]
        ds4.append(pltpu.async_copy(accum.at[pl.ds(rbase + b * K, K)],
                                    rows.at[base + 3], sem_g.at[base + 3]))
        loads[b] = ds4

    fire_block(0)
    sts = {}
    for b in range(nb):
        if b + 1 < nb and b + 1 not in loads:
            pass
        for d in loads[b]:
            d.wait()
        base = 4 * (b % 2)

        @pl.loop(0, K)
        def _combine(r):
            for c in range(HALF // 16):
                cs = pl.ds(c * 16, 16)
                acc = (rows[base + 0, r, cs] + rows[base + 1, r, cs]
                       + rows[base + 2, r, cs] + rows[base + 3, r, cs])
                rows[base + 0, r, cs] = acc * 0.25

        if b + 2 in sts:
            pass
        sts[b] = pltpu.async_copy(
            rows.at[base],
            light.at[pl.ds(cid * N_PAD + rbase + b * K, K)], sem_s.at[b % 2])
        if b + 1 < nb:
            if b - 1 in sts:
                sts[b - 1].wait()
            fire_block(b + 1)
    sts[nb - 1].wait()
    plsc.subcore_barrier()

    pltpu.sync_copy(upair.at[pl.ds(sid * U_PT, U_PT)],
                    srcv.at[pl.ds(0, U_PT)])
    for c in range(U_PT // 16):
        cs = pl.ds(c * 16, 16)
        srcv[cs] = srcv[cs] + cid * N_PAD
    pltpu.async_copy(light.at[srcv.at[pl.ds(0, U_PT)]],
                     rows.at[0, pl.ds(0, U_PT)], sem).wait()
    pltpu.sync_copy(rows.at[0, pl.ds(0, U_PT)],
                    uout.at[pl.ds(cid * BATCH_USERS + sid * U_PT, U_PT)])


@jax.jit
def _propagate(src2, dst2, w2, emb0, upair):
    mesh = plsc.VectorSubcoreMesh(core_axis_name="c", subcore_axis_name="s")
    return pl.kernel(
        _propagate_body,
        out_type=(
            jax.ShapeDtypeStruct((NC * N_PAD, HALF), jnp.float32),
            jax.ShapeDtypeStruct((NC * BATCH_USERS, HALF), jnp.float32),
            jax.ShapeDtypeStruct((NC * N_PAD, HALF), jnp.float32),
            jax.ShapeDtypeStruct((NC * N_PAD, HALF), jnp.float32),
        ),
        mesh=mesh,
        scratch_types=[
            pltpu.VMEM_SHARED((N_PAD, HALF), jnp.float32),
            pltpu.VMEM((K,), jnp.int32),
            pltpu.VMEM((SB, K), jnp.int32),
            pltpu.VMEM((SB, K), jnp.int32),
            pltpu.VMEM((SB, K), jnp.float32),
            pltpu.VMEM((K,), jnp.int32),
            pltpu.VMEM((K,), jnp.int32),
            pltpu.VMEM((K,), jnp.int32),
            pltpu.VMEM((K,), jnp.int32),
            pltpu.VMEM((K,), jnp.int32),
            pltpu.VMEM((K,), jnp.int32),
            pltpu.VMEM((K,), jnp.int32),
            pltpu.VMEM((K,), jnp.int32),
            pltpu.VMEM((K,), jnp.int32),
            pltpu.VMEM((K,), jnp.int32),
            pltpu.VMEM((K,), jnp.int32),
            pltpu.VMEM((K,), jnp.int32),
            pltpu.VMEM((K,), jnp.int32),
            pltpu.VMEM((K,), jnp.int32),
            pltpu.VMEM((K,), jnp.int32),
            pltpu.VMEM((K,), jnp.int32),
            pltpu.VMEM((SB, K, HALF), jnp.float32),
            pltpu.VMEM((K, HALF), jnp.float32),
            pltpu.SemaphoreType.DMA,
            pltpu.SemaphoreType.DMA,
            pltpu.SemaphoreType.DMA,
            pltpu.SemaphoreType.DMA((SB,)),
            pltpu.SemaphoreType.DMA((SB,)),
            pltpu.SemaphoreType.DMA,
        ],
        compiler_params=pltpu.CompilerParams(use_tc_tiling_on_sc=False),
    )(src2, dst2, w2, emb0, upair)


def _rating_body(u0, u1, i0, i1, out):
    acc = jax.lax.dot_general(u0[...], i0[...], (((1,), (1,)), ((), ())),
                              preferred_element_type=jnp.float32)
    acc += jax.lax.dot_general(u1[...], i1[...], (((1,), (1,)), ((), ())),
                               preferred_element_type=jnp.float32)
    out[...] = 1.0 / (1.0 + jnp.exp(-acc))


@jax.jit
def _rating(u0, u1, i0, i1):
    m_blk = 128
    grid = (BATCH_USERS // m_blk,)
    return pl.pallas_call(
        _rating_body,
        grid=grid,
        in_specs=[
            pl.BlockSpec((m_blk, HALF), lambda i: (i, 0)),
            pl.BlockSpec((m_blk, HALF), lambda i: (i, 0)),
            pl.BlockSpec((ITEMS_PAD, HALF), lambda i: (0, 0)),
            pl.BlockSpec((ITEMS_PAD, HALF), lambda i: (0, 0)),
        ],
        out_specs=pl.BlockSpec((m_blk, ITEMS_PAD), lambda i: (i, 0)),
        out_shape=jax.ShapeDtypeStruct((BATCH_USERS, ITEMS_PAD), jnp.float32),
    )(u0, u1, i0, i1)


def kernel(user_emb, item_emb, edge_weight, edge_index, users):
    all_emb = jnp.concatenate([user_emb, item_emb], axis=0)
    all_emb = jnp.pad(all_emb, ((0, N_PAD - N_NODES), (0, 0)))
    emb0 = all_emb.reshape(N_PAD, NC, HALF).transpose(1, 0, 2)
    emb0 = emb0.reshape(NC * N_PAD, HALF)

    src = edge_index[0]
    dst = edge_index[1]
    pad_e = E_PAD - N_EDGES
    pad_rows = N_NODES + (jnp.arange(pad_e, dtype=jnp.int32)
                          % (N_PAD - N_NODES))
    src_p = jnp.concatenate([src, pad_rows])
    dst_p = jnp.concatenate([dst, pad_rows])
    w_p = jnp.concatenate([edge_weight, jnp.zeros((pad_e,), jnp.float32)])
    src2 = src_p.reshape(-1, K)
    dst2 = dst_p.reshape(-1, K)
    w2 = w_p.reshape(-1, K)
    upair = users

    light, uout, _, _ = _propagate(src2, dst2, w2, emb0, upair)

    i0 = light[N_USERS:N_USERS + ITEMS_PAD]
    i1 = light[N_PAD + N_USERS:N_PAD + N_USERS + ITEMS_PAD]
    u0 = uout[:BATCH_USERS]
    u1 = uout[BATCH_USERS:]
    rating = _rating(u0, u1, i0, i1)
    return rating[:, :N_ITEMS]


# speedup vs baseline: 4.1993x; 1.0011x over previous
import jax
import jax.numpy as jnp
from jax import lax
from jax.experimental import pallas as pl
from jax.experimental.pallas import tpu as pltpu
from jax.experimental.pallas import tpu_sc as plsc

N_USERS = 4000
N_ITEMS = 6000
N_NODES = N_USERS + N_ITEMS
N_EDGES = 320000
DIM = 128
HALF = 64
N_LAYERS = 3
BATCH_USERS = 1024

NC = 2
NS = 16
K = 128

N_PAD = 10240
ROWS_PT = N_PAD // NS
SB = 8
NSB = 20
CHUNKS_PT = SB * NSB
E_PAD = NS * CHUNKS_PT * K
ITEMS_PAD = 6016
IROWS_PT = ITEMS_PAD // NS
U_PT = BATCH_USERS // NS


def _propagate_body(src2, dst2, w2, emb0, upair, light, uout, e1buf, e2buf,
                    accum, srcv, sidx, didx, wblk,
                    dv0, dv1, dv2, dv3, dv4, dv5, dv6, dv7,
                    sv0, sv1, sv2, sv3, sv4, sv5, sv6, sv7,
                    rows, stage, sem_ia, sem_da, sem_wa, sem_g, sem_s,
                    sem):
    cid = lax.axis_index("c")
    sid = lax.axis_index("s")
    rbase = sid * ROWS_PT
    ebase = sid * CHUNKS_PT

    def zero_stage():
        @pl.loop(0, K)
        def _z(r):
            for c in range(HALF // 16):
                stage[r, pl.ds(c * 16, 16)] = jnp.zeros((16,), jnp.float32)

    zero_stage()

    sources = [emb0, e1buf, e2buf]
    for layer in range(N_LAYERS):
        gather_src = sources[layer]
        za = [
            pltpu.async_copy(stage, accum.at[pl.ds(rbase + b * K, K)],
                             sem_g.at[b])
            for b in range(ROWS_PT // K)
        ]
        for d in za:
            d.wait()
        plsc.subcore_barrier()

        gsrc = gather_src
        dvs = [dv0, dv1, dv2, dv3, dv4, dv5, dv6, dv7]
        svs = [sv0, sv1, sv2, sv3, sv4, sv5, sv6, sv7]

        @pl.loop(0, NSB)
        def _superblock(t):
            row = ebase + t * SB
            i1 = pltpu.async_copy(src2.at[pl.ds(row, SB)], sidx, sem_ia)
            i2 = pltpu.async_copy(dst2.at[pl.ds(row, SB)], didx, sem_da)
            i3 = pltpu.async_copy(w2.at[pl.ds(row, SB)], wblk, sem_wa)
            i1.wait()
            off = cid * N_PAD
            gath = []
            for j in range(SB):
                for c in range(K // 16):
                    cs = pl.ds(c * 16, 16)
                    svs[j][cs] = sidx[j, cs] + off
                gath.append(
                    pltpu.async_copy(gsrc.at[svs[j]], rows.at[j],
                                     sem_g.at[j]))
            i2.wait()
            i3.wait()
            scat = []
            for j in range(SB):
                for c in range(K // 16):
                    cs = pl.ds(c * 16, 16)
                    dvs[j][cs] = didx[j, cs]
                gath[j].wait()

                @pl.loop(0, K // 16, unroll=2)
                def _scale(gg):
                    w16 = wblk[j, pl.ds(gg * 16, 16)]
                    for tt in range(16):
                        r = gg * 16 + tt
                        wt = w16[tt]
                        for c in range(HALF // 16):
                            sl = pl.ds(c * 16, 16)
                            rows[j, r, sl] = rows[j, r, sl] * wt

                scat.append(
                    pltpu.async_copy(rows.at[j], accum.at[dvs[j]],
                                     sem_s.at[j], add=True))
            for d in scat:
                d.wait()

        plsc.subcore_barrier()

        if layer < N_LAYERS - 1:
            nxt = sources[layer + 1]
            nb = ROWS_PT // K
            lds = [
                pltpu.async_copy(
                    accum.at[pl.ds(rbase + b * K, K)], rows.at[b],
                    sem_g.at[b])
                for b in range(nb)
            ]
            sts = []
            for b in range(nb):
                lds[b].wait()
                sts.append(pltpu.async_copy(
                    rows.at[b],
                    nxt.at[pl.ds(cid * N_PAD + rbase + b * K, K)],
                    sem_s.at[b]))
            for d in sts:
                d.wait()
            plsc.subcore_barrier()

    nb = ROWS_PT // K
    srcs4 = [emb0, e1buf, e2buf]
    loads = {}

    def fire_block(b):
        base = 4 * (b % 2)
        hsl = pl.ds(cid * N_PAD + rbase + b * K, K)
        ds4 = [pltpu.async_copy(srcs4[i].at[hsl], rows.at[base + i],
                                sem_g.at[base + i]) for i in range(3)]
        ds4.append(pltpu.async_copy(accum.at[pl.ds(rbase + b * K, K)],
                                    rows.at[base + 3], sem_g.at[base + 3]))
        loads[b] = ds4

    fire_block(0)
    sts = {}
    for b in range(nb):
        if b + 1 < nb and b + 1 not in loads:
            pass
        for d in loads[b]:
            d.wait()
        base = 4 * (b % 2)

        @pl.loop(0, K)
        def _combine(r):
            for c in range(HALF // 16):
                cs = pl.ds(c * 16, 16)
                acc = (rows[base + 0, r, cs] + rows[base + 1, r, cs]
                       + rows[base + 2, r, cs] + rows[base + 3, r, cs])
                rows[base + 0, r, cs] = acc * 0.25

        if b + 2 in sts:
            pass
        sts[b] = pltpu.async_copy(
            rows.at[base],
            light.at[pl.ds(cid * N_PAD + rbase + b * K, K)], sem_s.at[b % 2])
        if b + 1 < nb:
            if b - 1 in sts:
                sts[b - 1].wait()
            fire_block(b + 1)
    sts[nb - 1].wait()
    plsc.subcore_barrier()

    pltpu.sync_copy(upair.at[pl.ds(sid * U_PT, U_PT)],
                    srcv.at[pl.ds(0, U_PT)])
    for c in range(U_PT // 16):
        cs = pl.ds(c * 16, 16)
        srcv[cs] = srcv[cs] + cid * N_PAD
    pltpu.async_copy(light.at[srcv.at[pl.ds(0, U_PT)]],
                     rows.at[0, pl.ds(0, U_PT)], sem).wait()
    pltpu.sync_copy(rows.at[0, pl.ds(0, U_PT)],
                    uout.at[pl.ds(cid * BATCH_USERS + sid * U_PT, U_PT)])


@jax.jit
def _propagate(src2, dst2, w2, emb0, upair):
    mesh = plsc.VectorSubcoreMesh(core_axis_name="c", subcore_axis_name="s")
    return pl.kernel(
        _propagate_body,
        out_type=(
            jax.ShapeDtypeStruct((NC * N_PAD, HALF), jnp.float32),
            jax.ShapeDtypeStruct((NC * BATCH_USERS, HALF), jnp.float32),
            jax.ShapeDtypeStruct((NC * N_PAD, HALF), jnp.float32),
            jax.ShapeDtypeStruct((NC * N_PAD, HALF), jnp.float32),
        ),
        mesh=mesh,
        scratch_types=[
            pltpu.VMEM_SHARED((N_PAD, HALF), jnp.float32),
            pltpu.VMEM((K,), jnp.int32),
            pltpu.VMEM((SB, K), jnp.int32),
            pltpu.VMEM((SB, K), jnp.int32),
            pltpu.VMEM((SB, K), jnp.float32),
            pltpu.VMEM((K,), jnp.int32),
            pltpu.VMEM((K,), jnp.int32),
            pltpu.VMEM((K,), jnp.int32),
            pltpu.VMEM((K,), jnp.int32),
            pltpu.VMEM((K,), jnp.int32),
            pltpu.VMEM((K,), jnp.int32),
            pltpu.VMEM((K,), jnp.int32),
            pltpu.VMEM((K,), jnp.int32),
            pltpu.VMEM((K,), jnp.int32),
            pltpu.VMEM((K,), jnp.int32),
            pltpu.VMEM((K,), jnp.int32),
            pltpu.VMEM((K,), jnp.int32),
            pltpu.VMEM((K,), jnp.int32),
            pltpu.VMEM((K,), jnp.int32),
            pltpu.VMEM((K,), jnp.int32),
            pltpu.VMEM((K,), jnp.int32),
            pltpu.VMEM((SB, K, HALF), jnp.float32),
            pltpu.VMEM((K, HALF), jnp.float32),
            pltpu.SemaphoreType.DMA,
            pltpu.SemaphoreType.DMA,
            pltpu.SemaphoreType.DMA,
            pltpu.SemaphoreType.DMA((SB,)),
            pltpu.SemaphoreType.DMA((SB,)),
            pltpu.SemaphoreType.DMA,
        ],
        compiler_params=pltpu.CompilerParams(use_tc_tiling_on_sc=False),
    )(src2, dst2, w2, emb0, upair)


def _rating_body(u0, u1, i0, i1, out):
    acc = jax.lax.dot_general(u0[...], i0[...], (((1,), (1,)), ((), ())),
                              preferred_element_type=jnp.float32)
    acc += jax.lax.dot_general(u1[...], i1[...], (((1,), (1,)), ((), ())),
                               preferred_element_type=jnp.float32)
    out[...] = 1.0 / (1.0 + jnp.exp(-acc))


@jax.jit
def _rating(u0, u1, i0, i1):
    m_blk = 128
    grid = (BATCH_USERS // m_blk,)
    return pl.pallas_call(
        _rating_body,
        grid=grid,
        in_specs=[
            pl.BlockSpec((m_blk, HALF), lambda i: (i, 0)),
            pl.BlockSpec((m_blk, HALF), lambda i: (i, 0)),
            pl.BlockSpec((ITEMS_PAD, HALF), lambda i: (0, 0)),
            pl.BlockSpec((ITEMS_PAD, HALF), lambda i: (0, 0)),
        ],
        out_specs=pl.BlockSpec((m_blk, ITEMS_PAD), lambda i: (i, 0)),
        out_shape=jax.ShapeDtypeStruct((BATCH_USERS, ITEMS_PAD), jnp.float32),
    )(u0, u1, i0, i1)


def kernel(user_emb, item_emb, edge_weight, edge_index, users):
    all_emb = jnp.concatenate([user_emb, item_emb], axis=0)
    all_emb = jnp.pad(all_emb, ((0, N_PAD - N_NODES), (0, 0)))
    emb0 = all_emb.reshape(N_PAD, NC, HALF).transpose(1, 0, 2)
    emb0 = emb0.reshape(NC * N_PAD, HALF)

    src = edge_index[0]
    dst = edge_index[1]
    pad_e = E_PAD - N_EDGES
    pad_rows = N_NODES + (jnp.arange(pad_e, dtype=jnp.int32)
                          % (N_PAD - N_NODES))
    src_p = jnp.concatenate([src, pad_rows])
    dst_p = jnp.concatenate([dst, pad_rows])
    w_p = jnp.concatenate([edge_weight, jnp.zeros((pad_e,), jnp.float32)])
    src2 = src_p.reshape(-1, K)
    dst2 = dst_p.reshape(-1, K)
    w2 = w_p.reshape(-1, K)
    upair = users

    light, uout, _, _ = _propagate(src2, dst2, w2, emb0, upair)

    i0 = light[N_USERS:N_USERS + ITEMS_PAD]
    i1 = light[N_PAD + N_USERS:N_PAD + N_USERS + ITEMS_PAD]
    u0 = uout[:BATCH_USERS]
    u1 = uout[BATCH_USERS:]
    rating = _rating(u0, u1, i0, i1)
    return rating[:, :N_ITEMS]
